# Initial kernel scaffold; baseline (speedup 1.0000x reference)
import functools

import jax
import jax.numpy as jnp
import numpy as np
from jax import lax
from jax.experimental import pallas as pl
from jax.experimental.pallas import tpu as pltpu
from jax.experimental.pallas import tpu_sc as plsc

N = 10000
E = 160000
MSG = 32
H = 8
QD = MSG * H
SRCW = QD + MSG
HALF = 144

NC = 2
NS = 16
NW = NC * NS

EB = 2000
NBLK = 1000
NACC = 10240

GCHUNK = 128
NGCH = E // GCHUNK
SCHUNK = 80
NSCH_PER_SUB = E // (NS * SCHUNK)

_f32 = jnp.float32


def _k1_body(node_ref, wqs_ref, wd_ref, tsrc_ref, tdst_ref):
    x = node_ref[...]
    mu = jnp.mean(x, axis=1, keepdims=True)
    xc = x - mu
    var = jnp.mean(xc * xc, axis=1, keepdims=True)
    nn = xc * lax.rsqrt(var + 1e-5)
    tsrc_ref[...] = jnp.dot(nn, wqs_ref[...], preferred_element_type=_f32)
    tdst_ref[...] = jnp.dot(nn, wd_ref[...], preferred_element_type=_f32)


def _k2_body(rbf_ref, rsh_ref, gsrc_ref, gdst_ref, a16_ref, b8_ref, wrbf2_ref,
             wkvk_ref, shead_ref, coup_ref, logit_ref, gmax_ref):
    i = pl.program_id(0)
    rep = jnp.dot(rbf_ref[...], a16_ref[...], preferred_element_type=_f32)
    til = jnp.dot(rsh_ref[...], b8_ref[...], preferred_element_type=_f32)
    cpre = jnp.dot(rep * til, wrbf2_ref[...], preferred_element_type=_f32)
    gsrc = gsrc_ref[...]
    q = gsrc[:, :QD]
    s = gsrc[:, QD:SRCW]
    x1 = s + gdst_ref[...]
    coup = x1 * cpre * np.float32(1.0 / np.sqrt(8.0))
    coup_ref[...] = coup
    key = jnp.dot(coup, wkvk_ref[...], preferred_element_type=_f32)
    logits = jnp.dot(q * key, shead_ref[...], preferred_element_type=_f32)
    logits = logits * np.float32(1.0 / np.sqrt(32.0))
    logit_ref[...] = logits
    m = jnp.max(logits, axis=0, keepdims=True)

    @pl.when(i == 0)
    def _():
        gmax_ref[...] = m

    @pl.when(i != 0)
    def _():
        gmax_ref[...] = jnp.maximum(gmax_ref[...], m)


def _k3_body(coup_ref, logit_ref, gmax_ref, wkvv_ref, bhead_ref, contrib_ref):
    ex = jnp.exp(logit_ref[...] - gmax_ref[...])
    val = jnp.dot(coup_ref[...], wkvv_ref[...], preferred_element_type=_f32)
    wv = jnp.dot(ex, bhead_ref[...], preferred_element_type=_f32) * val
    zero = jnp.zeros((wv.shape[0], 12), _f32)
    contrib_ref[0] = jnp.concatenate([wv[:, :128], ex[:, :4], zero], axis=1)
    contrib_ref[1] = jnp.concatenate([wv[:, 128:], ex[:, 4:], zero], axis=1)


def _k4_body(acc_ref, node_ref, wmsg_ref, bhead_ref, out_ref):
    a = acc_ref[0]
    b = acc_ref[1]
    numer = jnp.concatenate([a[:, :128], b[:, :128]], axis=1)
    den8 = jnp.concatenate([a[:, 128:132], b[:, 128:132]], axis=1)
    d256 = jnp.dot(den8, bhead_ref[...], preferred_element_type=_f32) + 1e-16
    msg = jnp.dot(numer / d256, wmsg_ref[...], preferred_element_type=_f32)
    out_ref[...] = node_ref[...] + msg


def _s1_body(tsrc, tdst, sidx, didx, gsrc, gdst, idx1, idx2, rows_s, rows_d):
    wid = lax.axis_index("s") * NC + lax.axis_index("c")

    @pl.loop(0, 40)
    def _(k):
        cid = wid + k * NW

        @pl.when(cid < NGCH)
        def _():
            off = cid * GCHUNK
            pltpu.sync_copy(sidx.at[pl.ds(off, GCHUNK)], idx1)
            pltpu.sync_copy(tsrc.at[idx1], rows_s)
            pltpu.sync_copy(rows_s, gsrc.at[pl.ds(off, GCHUNK)])
            pltpu.sync_copy(didx.at[pl.ds(off, GCHUNK)], idx2)
            pltpu.sync_copy(tdst.at[idx2], rows_d)
            pltpu.sync_copy(rows_d, gdst.at[pl.ds(off, GCHUNK)])


def _s2_body(contrib, didx, out, acc, zbuf, idxv, rowsv):
    c = lax.axis_index("c")
    t = lax.axis_index("s")

    @pl.loop(0, 320)
    def _(r):
        @pl.loop(0, HALF // 16)
        def _(j):
            zbuf[r, pl.ds(j * 16, 16)] = jnp.zeros((16,), _f32)

    pltpu.sync_copy(zbuf, acc.at[pl.ds(t * 640, 320)])
    pltpu.sync_copy(zbuf, acc.at[pl.ds(t * 640 + 320, 320)])
    plsc.subcore_barrier()

    base = t * (E // NS)

    @pl.loop(0, NSCH_PER_SUB)
    def _(k):
        off = base + k * SCHUNK
        pltpu.sync_copy(didx.at[pl.ds(off, SCHUNK)], idxv)
        pltpu.sync_copy(contrib.at[c, pl.ds(off, SCHUNK)], rowsv)
        pltpu.sync_copy(rowsv, acc.at[idxv], add=True)

    plsc.subcore_barrier()
    pltpu.sync_copy(acc.at[pl.ds(t * 640, 640)], out.at[c, pl.ds(t * 640, 640)])


def _constants():
    A16 = np.zeros((16, 128), np.float32)
    B8 = np.zeros((8, 128), np.float32)
    for i in range(16):
        for j in range(8):
            A16[i, i * 8 + j] = 1.0
            B8[j, i * 8 + j] = 1.0
    Shead = np.zeros((256, 8), np.float32)
    for h in range(8):
        Shead[h * 32:(h + 1) * 32, h] = 1.0
    return jnp.asarray(A16), jnp.asarray(B8), jnp.asarray(Shead), jnp.asarray(Shead.T)


_SC_MESH = plsc.VectorSubcoreMesh(core_axis_name="c", subcore_axis_name="s")

_s1_call = functools.partial(
    pl.kernel,
    mesh=_SC_MESH,
    out_type=[jax.ShapeDtypeStruct((E, SRCW), _f32),
              jax.ShapeDtypeStruct((E, MSG), _f32)],
    scratch_types=[pltpu.VMEM((GCHUNK,), jnp.int32),
                   pltpu.VMEM((GCHUNK,), jnp.int32),
                   pltpu.VMEM((GCHUNK, SRCW), _f32),
                   pltpu.VMEM((GCHUNK, MSG), _f32)],
)(_s1_body)

_s2_call = functools.partial(
    pl.kernel,
    mesh=_SC_MESH,
    out_type=jax.ShapeDtypeStruct((NC, NACC, HALF), _f32),
    scratch_types=[pltpu.VMEM_SHARED((NACC, HALF), _f32),
                   pltpu.VMEM((320, HALF), _f32),
                   pltpu.VMEM((SCHUNK,), jnp.int32),
                   pltpu.VMEM((SCHUNK, HALF), _f32)],
)(_s2_body)


def kernel(node, rbf, rsh, edge_index, Wq, Wsrc, Wdst, Wrbf, Wkv, Wmsg):
    a16, b8, shead, bhead = _constants()
    wqs = jnp.concatenate([Wq, Wsrc], axis=1)
    wrbf2 = Wrbf.reshape(16, MSG, 8).transpose(0, 2, 1).reshape(128, MSG)
    wkvk = Wkv[:, :QD]
    wkvv = Wkv[:, QD:]
    src_idx = edge_index[0]
    dst_idx = edge_index[1]

    tsrc, tdst = pl.pallas_call(
        _k1_body,
        grid=(N // NBLK,),
        in_specs=[pl.BlockSpec((NBLK, 128), lambda i: (i, 0)),
                  pl.BlockSpec((128, SRCW), lambda i: (0, 0)),
                  pl.BlockSpec((128, MSG), lambda i: (0, 0))],
        out_specs=[pl.BlockSpec((NBLK, SRCW), lambda i: (i, 0)),
                   pl.BlockSpec((NBLK, MSG), lambda i: (i, 0))],
        out_shape=[jax.ShapeDtypeStruct((N, SRCW), _f32),
                   jax.ShapeDtypeStruct((N, MSG), _f32)],
    )(node, wqs, Wdst)

    gsrc, gdst = _s1_call(tsrc, tdst, src_idx, dst_idx)

    coup, logits, gmax = pl.pallas_call(
        _k2_body,
        grid=(E // EB,),
        in_specs=[pl.BlockSpec((EB, 16), lambda i: (i, 0)),
                  pl.BlockSpec((EB, 8), lambda i: (i, 0)),
                  pl.BlockSpec((EB, SRCW), lambda i: (i, 0)),
                  pl.BlockSpec((EB, MSG), lambda i: (i, 0)),
                  pl.BlockSpec((16, 128), lambda i: (0, 0)),
                  pl.BlockSpec((8, 128), lambda i: (0, 0)),
                  pl.BlockSpec((128, MSG), lambda i: (0, 0)),
                  pl.BlockSpec((MSG, QD), lambda i: (0, 0)),
                  pl.BlockSpec((QD, 8), lambda i: (0, 0))],
        out_specs=[pl.BlockSpec((EB, MSG), lambda i: (i, 0)),
                   pl.BlockSpec((EB, 8), lambda i: (i, 0)),
                   pl.BlockSpec((1, 8), lambda i: (0, 0))],
        out_shape=[jax.ShapeDtypeStruct((E, MSG), _f32),
                   jax.ShapeDtypeStruct((E, 8), _f32),
                   jax.ShapeDtypeStruct((1, 8), _f32)],
    )(rbf, rsh, gsrc, gdst, a16, b8, wrbf2, wkvk, shead)

    contrib = pl.pallas_call(
        _k3_body,
        grid=(E // EB,),
        in_specs=[pl.BlockSpec((EB, MSG), lambda i: (i, 0)),
                  pl.BlockSpec((EB, 8), lambda i: (i, 0)),
                  pl.BlockSpec((1, 8), lambda i: (0, 0)),
                  pl.BlockSpec((MSG, QD), lambda i: (0, 0)),
                  pl.BlockSpec((8, QD), lambda i: (0, 0))],
        out_specs=pl.BlockSpec((2, EB, HALF), lambda i: (0, i, 0)),
        out_shape=jax.ShapeDtypeStruct((2, E, HALF), _f32),
    )(coup, logits, gmax, wkvv, bhead)

    acc = _s2_call(contrib, dst_idx)

    out = pl.pallas_call(
        _k4_body,
        grid=(N // NBLK,),
        in_specs=[pl.BlockSpec((2, NBLK, HALF), lambda i: (0, i, 0)),
                  pl.BlockSpec((NBLK, 128), lambda i: (i, 0)),
                  pl.BlockSpec((QD, 128), lambda i: (0, 0)),
                  pl.BlockSpec((8, QD), lambda i: (0, 0))],
        out_specs=pl.BlockSpec((NBLK, 128), lambda i: (i, 0)),
        out_shape=jax.ShapeDtypeStruct((N, 128), _f32),
    )(acc, node, Wmsg, bhead)

    return out


# speedup vs baseline: 13.7595x; 13.7595x over previous
import functools

import jax
import jax.numpy as jnp
import numpy as np
from jax import lax
from jax.experimental import pallas as pl
from jax.experimental.pallas import tpu as pltpu
from jax.experimental.pallas import tpu_sc as plsc

N = 10000
E = 160000
MSG = 32
H = 8
QD = MSG * H
SRCW = QD + MSG
HALF = 144

NC = 2
NS = 16
NW = NC * NS

EB = 2000
NBLK = 1000
NACC = 10240

GCHUNK = 128
NGCH = E // GCHUNK
SCHUNK = 80
NSCH_PER_SUB = E // (NS * SCHUNK)

_f32 = jnp.float32


def _k1_body(node_ref, wqs_ref, wd_ref, tsrc_ref, tdst_ref):
    x = node_ref[...]
    mu = jnp.mean(x, axis=1, keepdims=True)
    xc = x - mu
    var = jnp.mean(xc * xc, axis=1, keepdims=True)
    nn = xc * lax.rsqrt(var + 1e-5)
    tsrc_ref[...] = jnp.dot(nn, wqs_ref[...], preferred_element_type=_f32)
    tdst_ref[...] = jnp.dot(nn, wd_ref[...], preferred_element_type=_f32)


def _k2_body(rbf_ref, rsh_ref, gsrc_ref, gdst_ref, a16_ref, b8_ref, wrbf2_ref,
             wkvk_ref, shead_ref, coup_ref, logit_ref, gmax_ref):
    i = pl.program_id(0)
    rep = jnp.dot(rbf_ref[...], a16_ref[...], preferred_element_type=_f32)
    til = jnp.dot(rsh_ref[...], b8_ref[...], preferred_element_type=_f32)
    cpre = jnp.dot(rep * til, wrbf2_ref[...], preferred_element_type=_f32)
    gsrc = gsrc_ref[...]
    q = gsrc[:, :QD]
    s = gsrc[:, QD:SRCW]
    x1 = s + gdst_ref[...]
    coup = x1 * cpre * np.float32(1.0 / np.sqrt(8.0))
    coup_ref[...] = coup
    key = jnp.dot(coup, wkvk_ref[...], preferred_element_type=_f32)
    logits = jnp.dot(q * key, shead_ref[...], preferred_element_type=_f32)
    logits = logits * np.float32(1.0 / np.sqrt(32.0))
    logit_ref[...] = logits
    m = jnp.max(logits, axis=0, keepdims=True)

    @pl.when(i == 0)
    def _():
        gmax_ref[...] = m

    @pl.when(i != 0)
    def _():
        gmax_ref[...] = jnp.maximum(gmax_ref[...], m)


def _k3_body(coup_ref, logit_ref, gmax_ref, wkvv_ref, bhead_ref, contrib_ref):
    ex = jnp.exp(logit_ref[...] - gmax_ref[...])
    val = jnp.dot(coup_ref[...], wkvv_ref[...], preferred_element_type=_f32)
    wv = jnp.dot(ex, bhead_ref[...], preferred_element_type=_f32) * val
    zero = jnp.zeros((wv.shape[0], 12), _f32)
    contrib_ref[0] = jnp.concatenate([wv[:, :128], ex[:, :4], zero], axis=1)
    contrib_ref[1] = jnp.concatenate([wv[:, 128:], ex[:, 4:], zero], axis=1)


def _k4_body(acc_ref, node_ref, wmsg_ref, bhead_ref, out_ref):
    a = acc_ref[0]
    b = acc_ref[1]
    numer = jnp.concatenate([a[:, :128], b[:, :128]], axis=1)
    den8 = jnp.concatenate([a[:, 128:132], b[:, 128:132]], axis=1)
    d256 = jnp.dot(den8, bhead_ref[...], preferred_element_type=_f32) + 1e-16
    msg = jnp.dot(numer / d256, wmsg_ref[...], preferred_element_type=_f32)
    out_ref[...] = node_ref[...] + msg


def _s1_body(tsrc, tdst, sidx, didx, gsrc, gdst, idx1, idx2, rows_s, rows_d):
    wid = lax.axis_index("s") * NC + lax.axis_index("c")

    @pl.loop(0, 40)
    def _(k):
        cid = wid + k * NW

        @pl.when(cid < NGCH)
        def _():
            off = cid * GCHUNK
            pltpu.sync_copy(sidx.at[pl.ds(off, GCHUNK)], idx1)
            pltpu.sync_copy(tsrc.at[idx1], rows_s)
            pltpu.sync_copy(rows_s, gsrc.at[pl.ds(off, GCHUNK)])
            pltpu.sync_copy(didx.at[pl.ds(off, GCHUNK)], idx2)
            pltpu.sync_copy(tdst.at[idx2], rows_d)
            pltpu.sync_copy(rows_d, gdst.at[pl.ds(off, GCHUNK)])


def _s2_body(contrib, didx, out, acc, idxv, rowsv):
    c = lax.axis_index("c")
    t = lax.axis_index("s")

    @pl.loop(0, SCHUNK)
    def _(r):
        @pl.loop(0, HALF // 16)
        def _(j):
            rowsv[r, pl.ds(j * 16, 16)] = jnp.zeros((16,), _f32)

    @pl.loop(0, 640 // SCHUNK)
    def _(z):
        pltpu.sync_copy(rowsv, acc.at[pl.ds(t * 640 + z * SCHUNK, SCHUNK)])

    plsc.subcore_barrier()

    base = t * (E // NS)

    @pl.loop(0, NSCH_PER_SUB)
    def _(k):
        off = base + k * SCHUNK
        pltpu.sync_copy(didx.at[pl.ds(off, SCHUNK)], idxv)
        pltpu.sync_copy(contrib.at[c, pl.ds(off, SCHUNK)], rowsv)
        pltpu.sync_copy(rowsv, acc.at[idxv], add=True)

    plsc.subcore_barrier()
    pltpu.sync_copy(acc.at[pl.ds(t * 640, 640)], out.at[c, pl.ds(t * 640, 640)])


def _constants():
    A16 = np.zeros((16, 128), np.float32)
    B8 = np.zeros((8, 128), np.float32)
    for i in range(16):
        for j in range(8):
            A16[i, i * 8 + j] = 1.0
            B8[j, i * 8 + j] = 1.0
    Shead = np.zeros((256, 8), np.float32)
    for h in range(8):
        Shead[h * 32:(h + 1) * 32, h] = 1.0
    return jnp.asarray(A16), jnp.asarray(B8), jnp.asarray(Shead), jnp.asarray(Shead.T)


_SC_MESH = plsc.VectorSubcoreMesh(core_axis_name="c", subcore_axis_name="s")
_SC_PARAMS = pltpu.CompilerParams(use_tc_tiling_on_sc=False)

_s1_call = functools.partial(
    pl.kernel,
    mesh=_SC_MESH,
    compiler_params=_SC_PARAMS,
    out_type=[jax.ShapeDtypeStruct((E, SRCW), _f32),
              jax.ShapeDtypeStruct((E, MSG), _f32)],
    scratch_types=[pltpu.VMEM((GCHUNK,), jnp.int32),
                   pltpu.VMEM((GCHUNK,), jnp.int32),
                   pltpu.VMEM((GCHUNK, SRCW), _f32),
                   pltpu.VMEM((GCHUNK, MSG), _f32)],
)(_s1_body)

_s2_call = functools.partial(
    pl.kernel,
    mesh=_SC_MESH,
    compiler_params=_SC_PARAMS,
    out_type=jax.ShapeDtypeStruct((NC, NACC, HALF), _f32),
    scratch_types=[pltpu.VMEM_SHARED((NACC, HALF), _f32),
                   pltpu.VMEM((SCHUNK,), jnp.int32),
                   pltpu.VMEM((SCHUNK, HALF), _f32)],
)(_s2_body)


def kernel(node, rbf, rsh, edge_index, Wq, Wsrc, Wdst, Wrbf, Wkv, Wmsg):
    a16, b8, shead, bhead = _constants()
    wqs = jnp.concatenate([Wq, Wsrc], axis=1)
    wrbf2 = Wrbf.reshape(16, MSG, 8).transpose(0, 2, 1).reshape(128, MSG)
    wkvk = Wkv[:, :QD]
    wkvv = Wkv[:, QD:]
    src_idx = edge_index[0]
    dst_idx = edge_index[1]

    tsrc, tdst = pl.pallas_call(
        _k1_body,
        grid=(N // NBLK,),
        in_specs=[pl.BlockSpec((NBLK, 128), lambda i: (i, 0)),
                  pl.BlockSpec((128, SRCW), lambda i: (0, 0)),
                  pl.BlockSpec((128, MSG), lambda i: (0, 0))],
        out_specs=[pl.BlockSpec((NBLK, SRCW), lambda i: (i, 0)),
                   pl.BlockSpec((NBLK, MSG), lambda i: (i, 0))],
        out_shape=[jax.ShapeDtypeStruct((N, SRCW), _f32),
                   jax.ShapeDtypeStruct((N, MSG), _f32)],
    )(node, wqs, Wdst)

    gsrc, gdst = _s1_call(tsrc, tdst, src_idx, dst_idx)

    coup, logits, gmax = pl.pallas_call(
        _k2_body,
        grid=(E // EB,),
        in_specs=[pl.BlockSpec((EB, 16), lambda i: (i, 0)),
                  pl.BlockSpec((EB, 8), lambda i: (i, 0)),
                  pl.BlockSpec((EB, SRCW), lambda i: (i, 0)),
                  pl.BlockSpec((EB, MSG), lambda i: (i, 0)),
                  pl.BlockSpec((16, 128), lambda i: (0, 0)),
                  pl.BlockSpec((8, 128), lambda i: (0, 0)),
                  pl.BlockSpec((128, MSG), lambda i: (0, 0)),
                  pl.BlockSpec((MSG, QD), lambda i: (0, 0)),
                  pl.BlockSpec((QD, 8), lambda i: (0, 0))],
        out_specs=[pl.BlockSpec((EB, MSG), lambda i: (i, 0)),
                   pl.BlockSpec((EB, 8), lambda i: (i, 0)),
                   pl.BlockSpec((1, 8), lambda i: (0, 0))],
        out_shape=[jax.ShapeDtypeStruct((E, MSG), _f32),
                   jax.ShapeDtypeStruct((E, 8), _f32),
                   jax.ShapeDtypeStruct((1, 8), _f32)],
    )(rbf, rsh, gsrc, gdst, a16, b8, wrbf2, wkvk, shead)

    contrib = pl.pallas_call(
        _k3_body,
        grid=(E // EB,),
        in_specs=[pl.BlockSpec((EB, MSG), lambda i: (i, 0)),
                  pl.BlockSpec((EB, 8), lambda i: (i, 0)),
                  pl.BlockSpec((1, 8), lambda i: (0, 0)),
                  pl.BlockSpec((MSG, QD), lambda i: (0, 0)),
                  pl.BlockSpec((8, QD), lambda i: (0, 0))],
        out_specs=pl.BlockSpec((2, EB, HALF), lambda i: (0, i, 0)),
        out_shape=jax.ShapeDtypeStruct((2, E, HALF), _f32),
    )(coup, logits, gmax, wkvv, bhead)

    acc = _s2_call(contrib, dst_idx)

    out = pl.pallas_call(
        _k4_body,
        grid=(N // NBLK,),
        in_specs=[pl.BlockSpec((2, NBLK, HALF), lambda i: (0, i, 0)),
                  pl.BlockSpec((NBLK, 128), lambda i: (i, 0)),
                  pl.BlockSpec((QD, 128), lambda i: (0, 0)),
                  pl.BlockSpec((8, QD), lambda i: (0, 0))],
        out_specs=pl.BlockSpec((NBLK, 128), lambda i: (i, 0)),
        out_shape=jax.ShapeDtypeStruct((N, 128), _f32),
    )(acc, node, Wmsg, bhead)

    return out
